# baseline (device time: 14970 ns/iter reference)
import jax
import jax.numpy as jnp
from jax import lax
from jax.experimental import pallas as pl
from jax.experimental.pallas import tpu as pltpu

N_DEV = 4


def kernel(A, B):
    m_per, k = A.shape
    _, n = B.shape

    def body(a_ref, b_ref, out_hbm, a_f32_ref, b_f32_ref, a_bf_ref,
             b_bf_ref, comm_ref, blk_ref, in_sems, send_sems, recv_sems,
             copy_sems):
        my = lax.axis_index("i")

        cp_a = pltpu.make_async_copy(a_ref, a_f32_ref, in_sems.at[0])
        cp_a.start()
        cp_b = pltpu.make_async_copy(b_ref, b_f32_ref, in_sems.at[1])
        cp_b.start()

        barrier = pltpu.get_barrier_semaphore()
        for d in range(1, N_DEV):
            pl.semaphore_signal(
                barrier, inc=1,
                device_id=((my + d) % N_DEV,),
                device_id_type=pl.DeviceIdType.MESH,
            )
        pl.semaphore_wait(barrier, N_DEV - 1)

        cp_a.wait()
        a_bf_ref[...] = a_f32_ref[...].astype(jnp.bfloat16)

        rdmas = []
        for d in range(1, N_DEV):
            rdma = pltpu.make_async_remote_copy(
                src_ref=a_bf_ref,
                dst_ref=comm_ref.at[d - 1],
                send_sem=send_sems.at[d - 1],
                recv_sem=recv_sems.at[d - 1],
                device_id=((my + d) % N_DEV,),
                device_id_type=pl.DeviceIdType.MESH,
            )
            rdma.start()
            rdmas.append(rdma)

        cp_b.wait()
        b_bf_ref[...] = b_f32_ref[...].astype(jnp.bfloat16)

        copies = []

        def compute_block(slot, a_chunk, origin):
            blk_ref[slot] = jnp.dot(
                a_chunk, b_bf_ref[...], preferred_element_type=jnp.float32
            ).astype(jnp.bfloat16)
            cp = pltpu.make_async_copy(
                blk_ref.at[slot],
                out_hbm.at[pl.ds(origin * m_per, m_per), :],
                copy_sems.at[slot],
            )
            cp.start()
            copies.append(cp)

        compute_block(0, a_bf_ref[...], my)

        for slot, d in enumerate((1, 3, 2), start=1):
            rdmas[d - 1].wait_recv()
            compute_block(slot, comm_ref[d - 1], (my - d) % N_DEV)

        for cp in copies:
            cp.wait()
        for d in range(1, N_DEV):
            rdmas[d - 1].wait_send()

    return pl.pallas_call(
        body,
        out_shape=jax.ShapeDtypeStruct((N_DEV * m_per, n), jnp.bfloat16),
        in_specs=[
            pl.BlockSpec(memory_space=pl.ANY),
            pl.BlockSpec(memory_space=pl.ANY),
        ],
        out_specs=pl.BlockSpec(memory_space=pl.ANY),
        scratch_shapes=[
            pltpu.VMEM((m_per, k), jnp.float32),
            pltpu.VMEM((k, n), jnp.float32),
            pltpu.VMEM((m_per, k), jnp.bfloat16),
            pltpu.VMEM((k, n), jnp.bfloat16),
            pltpu.VMEM((N_DEV - 1, m_per, k), jnp.bfloat16),
            pltpu.VMEM((N_DEV, m_per, n), jnp.bfloat16),
            pltpu.SemaphoreType.DMA((2,)),
            pltpu.SemaphoreType.DMA((N_DEV - 1,)),
            pltpu.SemaphoreType.DMA((N_DEV - 1,)),
            pltpu.SemaphoreType.DMA((N_DEV,)),
        ],
        compiler_params=pltpu.CompilerParams(collective_id=0),
    )(A, B)


# device time: 13730 ns/iter; 1.0903x vs baseline; 1.0903x over previous
import jax
import jax.numpy as jnp
from jax import lax
from jax.experimental import pallas as pl
from jax.experimental.pallas import tpu as pltpu

N_DEV = 4


def kernel(A, B):
    m_per, k = A.shape
    _, n = B.shape

    def body(a_ref, b_ref, out_hbm, a_f32_ref, b_f32_ref, a_bf_ref,
             b_bf_ref, comm_ref, blk_ref, in_sems, send_sems, recv_sems,
             copy_sems):
        my = lax.axis_index("i")

        cp_a = pltpu.make_async_copy(a_ref, a_f32_ref, in_sems.at[0])
        cp_a.start()
        cp_b = pltpu.make_async_copy(b_ref, b_f32_ref, in_sems.at[1])
        cp_b.start()

        barrier = pltpu.get_barrier_semaphore()
        for d in range(1, N_DEV):
            pl.semaphore_signal(
                barrier, inc=1,
                device_id=((my + d) % N_DEV,),
                device_id_type=pl.DeviceIdType.MESH,
            )
        pl.semaphore_wait(barrier, N_DEV - 1)

        cp_a.wait()
        a_bf_ref[...] = a_f32_ref[...].astype(jnp.bfloat16)

        rdmas = []
        for d in range(1, N_DEV):
            rdma = pltpu.make_async_remote_copy(
                src_ref=a_bf_ref,
                dst_ref=comm_ref.at[d - 1],
                send_sem=send_sems.at[d - 1],
                recv_sem=recv_sems.at[d - 1],
                device_id=((my + d) % N_DEV,),
                device_id_type=pl.DeviceIdType.MESH,
            )
            rdma.start()
            rdmas.append(rdma)

        cp_b.wait()
        b_bf_ref[...] = b_f32_ref[...].astype(jnp.bfloat16)

        copies = []

        def compute_block(slot, a_chunk, origin):
            blk_ref[slot] = jnp.dot(
                a_chunk, b_bf_ref[...], preferred_element_type=jnp.float32
            ).astype(jnp.bfloat16)
            cp = pltpu.make_async_copy(
                blk_ref.at[slot],
                out_hbm.at[pl.ds(origin * m_per, m_per), :],
                copy_sems.at[slot],
            )
            cp.start()
            copies.append(cp)

        compute_block(0, a_bf_ref[...], my)

        for slot, d in enumerate((1, 3, 2), start=1):
            rdmas[d - 1].wait_recv()
            compute_block(slot, comm_ref[d - 1], (my - d) % N_DEV)

        for cp in copies:
            cp.wait()
        for d in range(1, N_DEV):
            rdmas[d - 1].wait_send()

    return pl.pallas_call(
        body,
        out_shape=jax.ShapeDtypeStruct((N_DEV * m_per, n), jnp.bfloat16),
        in_specs=[
            pl.BlockSpec(memory_space=pltpu.MemorySpace.HBM),
            pl.BlockSpec(memory_space=pltpu.MemorySpace.HBM),
        ],
        out_specs=pl.BlockSpec(memory_space=pltpu.MemorySpace.HBM),
        scratch_shapes=[
            pltpu.VMEM((m_per, k), jnp.float32),
            pltpu.VMEM((k, n), jnp.float32),
            pltpu.VMEM((m_per, k), jnp.bfloat16),
            pltpu.VMEM((k, n), jnp.bfloat16),
            pltpu.VMEM((N_DEV - 1, m_per, k), jnp.bfloat16),
            pltpu.VMEM((N_DEV, m_per, n), jnp.bfloat16),
            pltpu.SemaphoreType.DMA((2,)),
            pltpu.SemaphoreType.DMA((N_DEV - 1,)),
            pltpu.SemaphoreType.DMA((N_DEV - 1,)),
            pltpu.SemaphoreType.DMA((N_DEV,)),
        ],
        compiler_params=pltpu.CompilerParams(collective_id=0),
    )(A, B)
